# baseline (speedup 1.0000x reference)
import functools

import jax
import jax.numpy as jnp
from jax import lax
from jax.experimental import pallas as pl
from jax.experimental.pallas import tpu as pltpu
from jax.experimental.pallas import tpu_sc as plsc

N = 10000
K = 16
D = 256
LANES = 16
DW = D // 2
GL = DW // LANES
NC = 2
NS = 16
NW = NC * NS
NPW = 320
NP = NW * NPW
C = 2
CK = C * K
CHUNKS = NPW // C
NBUF = 4
GROUPS = CHUNKS // NBUF

_FMT = plsc.PackFormat.INTERLEAVED


def _agg_body(x_hbm, idx_hbm, out_hbm, idx_v, rows_v, agg_v, gsem):
    wid = lax.axis_index("s") * NC + lax.axis_index("c")
    pltpu.sync_copy(idx_hbm.at[wid], idx_v)

    def issue(g, slot, b):
        pltpu.async_copy(
            x_hbm.at[idx_v.at[g, pl.ds(slot * CK, CK)]], rows_v.at[b], gsem)

    def wait(g, slot, b):
        pltpu.make_async_copy(
            x_hbm.at[idx_v.at[g, pl.ds(slot * CK, CK)]], rows_v.at[b], gsem).wait()

    def compute_chunk(c, b):
        def node_body(j, _):
            row0 = j * K
            node = c * C + j
            for t in range(GL):
                col = t * LANES

                def load(k):
                    v = rows_v[b, row0 + k, pl.ds(col, LANES)]
                    return plsc.unpack(plsc.bitcast(v, jnp.bfloat16), format=_FMT)

                sa, sb = load(0)
                for k in range(1, K):
                    pa, pb = load(k)
                    sa = sa + pa
                    sb = sb + pb
                packed = plsc.pack(sa, sb, format=_FMT)
                agg_v[node, pl.ds(col, LANES)] = plsc.bitcast(packed, jnp.int32)
            return 0

        lax.fori_loop(0, C, node_body, 0)

    for b in range(NBUF - 1):
        issue(0, b, b)

    def group_body(i, _):
        c0 = i * NBUF
        for b in range(NBUF):
            c = c0 + b
            wait(i, b, b)
            slot = (b + NBUF - 1) % NBUF
            issue(i if b == 0 else i + 1, slot, slot)
            compute_chunk(c, b)
        return 0

    lax.fori_loop(0, GROUPS - 1, group_body, 0)
    g = GROUPS - 1
    c0 = g * NBUF
    for b in range(NBUF):
        c = c0 + b
        wait(g, b, b)
        if b == 0:
            issue(g, NBUF - 1, NBUF - 1)
        compute_chunk(c, b)
    pltpu.sync_copy(agg_v, out_hbm.at[wid])


@functools.cache
def _agg_call():
    mesh = plsc.VectorSubcoreMesh(core_axis_name="c", subcore_axis_name="s")
    return pl.kernel(
        _agg_body,
        out_type=jax.ShapeDtypeStruct((NW, NPW, DW), jnp.int32),
        mesh=mesh,
        scratch_types=[
            pltpu.VMEM((GROUPS, NBUF * CK), jnp.int32),
            pltpu.VMEM((NBUF, CK, DW), jnp.int32),
            pltpu.VMEM((NPW, DW), jnp.int32),
            pltpu.SemaphoreType.DMA,
        ],
        compiler_params=pltpu.CompilerParams(needs_layout_passes=False),
    )


RT = 1000


def _mlp_body(eps_ref, x_ref, agg_ref, w1_ref, b1_ref, w2_ref, b2_ref, o_ref):
    agg = agg_ref[...].astype(jnp.float32)
    h = (1.0 + eps_ref[0]) * x_ref[...] + agg
    h1 = jnp.dot(h, w1_ref[...], preferred_element_type=jnp.float32) + b1_ref[...]
    h1 = jnp.maximum(h1, 0.0)
    o_ref[...] = jnp.dot(h1, w2_ref[...], preferred_element_type=jnp.float32) + b2_ref[...]


@functools.cache
def _mlp_call():
    return pl.pallas_call(
        _mlp_body,
        grid=(N // RT,),
        in_specs=[
            pl.BlockSpec(memory_space=pltpu.SMEM),
            pl.BlockSpec((RT, D), lambda i: (i, 0)),
            pl.BlockSpec((RT, D), lambda i: (i, 0)),
            pl.BlockSpec((D, D), lambda i: (0, 0)),
            pl.BlockSpec((1, D), lambda i: (0, 0)),
            pl.BlockSpec((D, D), lambda i: (0, 0)),
            pl.BlockSpec((1, D), lambda i: (0, 0)),
        ],
        out_specs=pl.BlockSpec((RT, D), lambda i: (i, 0)),
        out_shape=jax.ShapeDtypeStruct((N, D), jnp.float32),
    )


def kernel(x, neigh, eps, W1, b1, W2, b2):
    x2d = x[0]
    idx = neigh.astype(jnp.int32)
    idx = jnp.pad(idx, ((0, NP - N), (0, 0)))
    idx = idx.reshape(NW, GROUPS, NBUF * CK)
    x_bf = x2d.astype(jnp.bfloat16)
    xi = lax.bitcast_convert_type(x_bf.reshape(N, DW, 2), jnp.int32)
    agg_i = _agg_call()(xi, idx).reshape(NP, DW)
    agg = lax.bitcast_convert_type(agg_i, jnp.bfloat16).reshape(NP, D)
    eps_arr = jnp.reshape(eps, (1,)).astype(jnp.float32)
    out = _mlp_call()(eps_arr, x2d, agg, W1, jnp.reshape(b1, (1, D)),
                      W2, jnp.reshape(b2, (1, D)))
    return out[None]


# speedup vs baseline: 1.1510x; 1.1510x over previous
import functools

import jax
import jax.numpy as jnp
from jax import lax
from jax.experimental import pallas as pl
from jax.experimental.pallas import tpu as pltpu
from jax.experimental.pallas import tpu_sc as plsc

N = 10000
K = 16
D = 256
LANES = 16
DW = D // 2
GL = DW // LANES
NC = 2
NS = 16
NW = NC * NS
NPW = 320
NP = NW * NPW
C = 2
CK = C * K
CHUNKS = NPW // C
NBUF = 4
GROUPS = CHUNKS // NBUF

_FMT = plsc.PackFormat.INTERLEAVED


def _agg_body(x_hbm, idx_hbm, out_hbm, idx_v, rows_v, agg_v, gsem):
    wid = lax.axis_index("s") * NC + lax.axis_index("c")
    pltpu.sync_copy(idx_hbm.at[wid], idx_v)

    def issue(g, slot, b):
        pltpu.async_copy(
            x_hbm.at[idx_v.at[g, pl.ds(slot * CK, CK)]], rows_v.at[b], gsem)

    def wait(g, slot, b):
        pltpu.make_async_copy(
            x_hbm.at[idx_v.at[g, pl.ds(slot * CK, CK)]], rows_v.at[b], gsem).wait()

    def compute_chunk(c, b):
        def node_body(j, _):
            row0 = j * K
            node = c * C + j
            for t in range(GL):
                col = t * LANES

                def load(k):
                    v = rows_v[b, row0 + k, pl.ds(col, LANES)]
                    return plsc.unpack(plsc.bitcast(v, jnp.bfloat16), format=_FMT)

                sa, sb = load(0)
                for k in range(1, K):
                    pa, pb = load(k)
                    sa = sa + pa
                    sb = sb + pb
                packed = plsc.pack(sa, sb, format=_FMT)
                agg_v[node, pl.ds(col, LANES)] = plsc.bitcast(packed, jnp.int32)
            return 0

        lax.fori_loop(0, C, node_body, 0)

    for b in range(NBUF - 1):
        issue(0, b, b)

    def group_body(i, _):
        c0 = i * NBUF
        for b in range(NBUF):
            c = c0 + b
            wait(i, b, b)
            slot = (b + NBUF - 1) % NBUF
            issue(i if b == 0 else i + 1, slot, slot)
            compute_chunk(c, b)
        return 0

    lax.fori_loop(0, GROUPS - 1, group_body, 0)
    g = GROUPS - 1
    c0 = g * NBUF
    for b in range(NBUF):
        c = c0 + b
        wait(g, b, b)
        if b == 0:
            issue(g, NBUF - 1, NBUF - 1)
        compute_chunk(c, b)
    pltpu.sync_copy(agg_v, out_hbm.at[wid])


@functools.cache
def _agg_call():
    mesh = plsc.VectorSubcoreMesh(core_axis_name="c", subcore_axis_name="s")
    return pl.kernel(
        _agg_body,
        out_type=jax.ShapeDtypeStruct((NW, NPW, DW), jnp.int32),
        mesh=mesh,
        scratch_types=[
            pltpu.VMEM((GROUPS, NBUF * CK), jnp.int32),
            pltpu.VMEM((NBUF, CK, DW), jnp.int32),
            pltpu.VMEM((NPW, DW), jnp.int32),
            pltpu.SemaphoreType.DMA,
        ],
        compiler_params=pltpu.CompilerParams(needs_layout_passes=False),
    )


RT = 1000


def _mlp_body(eps_ref, x_ref, agg_ref, w1_ref, w1e_ref, w1o_ref, b1_ref,
              w2_ref, b2_ref, o_ref):
    w = agg_ref[...]
    lo = lax.bitcast_convert_type(w << 16, jnp.float32)
    hi = lax.bitcast_convert_type(w & jnp.int32(-65536), jnp.float32)
    h1 = (1.0 + eps_ref[0]) * jnp.dot(
        x_ref[...], w1_ref[...], preferred_element_type=jnp.float32)
    h1 = h1 + jnp.dot(lo, w1e_ref[...], preferred_element_type=jnp.float32)
    h1 = h1 + jnp.dot(hi, w1o_ref[...], preferred_element_type=jnp.float32)
    h1 = jnp.maximum(h1 + b1_ref[...], 0.0)
    o_ref[...] = jnp.dot(h1, w2_ref[...], preferred_element_type=jnp.float32) + b2_ref[...]


@functools.cache
def _mlp_call():
    return pl.pallas_call(
        _mlp_body,
        grid=(N // RT,),
        in_specs=[
            pl.BlockSpec(memory_space=pltpu.SMEM),
            pl.BlockSpec((RT, D), lambda i: (i, 0)),
            pl.BlockSpec((RT, DW), lambda i: (i, 0)),
            pl.BlockSpec((D, D), lambda i: (0, 0)),
            pl.BlockSpec((DW, D), lambda i: (0, 0)),
            pl.BlockSpec((DW, D), lambda i: (0, 0)),
            pl.BlockSpec((1, D), lambda i: (0, 0)),
            pl.BlockSpec((D, D), lambda i: (0, 0)),
            pl.BlockSpec((1, D), lambda i: (0, 0)),
        ],
        out_specs=pl.BlockSpec((RT, D), lambda i: (i, 0)),
        out_shape=jax.ShapeDtypeStruct((N, D), jnp.float32),
    )


def kernel(x, neigh, eps, W1, b1, W2, b2):
    x2d = x[0]
    idx = neigh.astype(jnp.int32)
    idx = jnp.pad(idx, ((0, NP - N), (0, 0)))
    idx = idx.reshape(NW, GROUPS, NBUF * CK)
    y = lax.bitcast_convert_type(x2d, jnp.uint32)
    r = (y + jnp.uint32(0x8000)) >> 16
    rp = r.reshape(N, DW, 2)
    xi = lax.bitcast_convert_type(rp[..., 0] | (rp[..., 1] << 16), jnp.int32)
    agg_i = _agg_call()(xi, idx).reshape(NP, DW)
    eps_arr = jnp.reshape(eps, (1,)).astype(jnp.float32)
    out = _mlp_call()(eps_arr, x2d, agg_i, W1, W1[0::2], W1[1::2],
                      jnp.reshape(b1, (1, D)), W2, jnp.reshape(b2, (1, D)))
    return out[None]


# speedup vs baseline: 1.4090x; 1.2241x over previous
import functools

import jax
import jax.numpy as jnp
from jax import lax
from jax.experimental import pallas as pl
from jax.experimental.pallas import tpu as pltpu
from jax.experimental.pallas import tpu_sc as plsc

N = 10000
K = 16
D = 256
LANES = 16
DW = D // 2
GL = DW // LANES
NC = 2
NS = 16
NW = NC * NS
NPW = 320
NP = NW * NPW
C = 2
CK = C * K
CHUNKS = NPW // C
NBUF = 4
GROUPS = CHUNKS // NBUF

_FMT = plsc.PackFormat.INTERLEAVED


def _agg_body(x_hbm, idx_hbm, out_hbm, idx_v, rows_v, agg_v, gsem):
    wid = lax.axis_index("s") * NC + lax.axis_index("c")
    pltpu.sync_copy(idx_hbm.at[wid], idx_v)

    def issue(g, slot, b):
        pltpu.async_copy(
            x_hbm.at[idx_v.at[g, pl.ds(slot * CK, CK)]], rows_v.at[b], gsem)

    def wait(g, slot, b):
        pltpu.make_async_copy(
            x_hbm.at[idx_v.at[g, pl.ds(slot * CK, CK)]], rows_v.at[b], gsem).wait()

    def compute_chunk(c, b):
        def node_body(j, _):
            row0 = j * K
            node = c * C + j
            for t in range(GL):
                col = t * LANES

                def load(k):
                    v = rows_v[b, row0 + k, pl.ds(col, LANES)]
                    return plsc.unpack(plsc.bitcast(v, jnp.bfloat16), format=_FMT)

                sa, sb = load(0)
                for k in range(1, K):
                    pa, pb = load(k)
                    sa = sa + pa
                    sb = sb + pb
                packed = plsc.pack(sa, sb, format=_FMT)
                agg_v[node, pl.ds(col, LANES)] = plsc.bitcast(packed, jnp.int32)
            return 0

        lax.fori_loop(0, C, node_body, 0)

    for b in range(NBUF - 1):
        issue(0, b, b)

    def group_body(i, _):
        c0 = i * NBUF
        for b in range(NBUF):
            c = c0 + b
            wait(i, b, b)
            slot = (b + NBUF - 1) % NBUF
            issue(i if b == 0 else i + 1, slot, slot)
            compute_chunk(c, b)
        return 0

    lax.fori_loop(0, GROUPS - 1, group_body, 0)
    g = GROUPS - 1
    c0 = g * NBUF
    for b in range(NBUF):
        c = c0 + b
        wait(g, b, b)
        if b == 0:
            issue(g, NBUF - 1, NBUF - 1)
        compute_chunk(c, b)
    pltpu.sync_copy(agg_v, out_hbm.at[wid])


@functools.cache
def _agg_call():
    mesh = plsc.VectorSubcoreMesh(core_axis_name="c", subcore_axis_name="s")
    return pl.kernel(
        _agg_body,
        out_type=jax.ShapeDtypeStruct((NW, NPW, DW), jnp.int32),
        mesh=mesh,
        scratch_types=[
            pltpu.VMEM((GROUPS, NBUF * CK), jnp.int32),
            pltpu.VMEM((NBUF, CK, DW), jnp.int32),
            pltpu.VMEM((NPW, DW), jnp.int32),
            pltpu.SemaphoreType.DMA,
        ],
        compiler_params=pltpu.CompilerParams(needs_layout_passes=False),
    )


RT = 1000


def _mlp_body(eps_ref, x_ref, agg_ref, w1_ref, w1e_ref, w1o_ref, b1_ref,
              w2_ref, b2_ref, o_ref):
    w = agg_ref[...]
    lo = lax.bitcast_convert_type(w << 16, jnp.float32)
    hi = lax.bitcast_convert_type(w & jnp.int32(-65536), jnp.float32)
    h1 = (1.0 + eps_ref[0]) * jnp.dot(
        x_ref[...], w1_ref[...], preferred_element_type=jnp.float32)
    h1 = h1 + jnp.dot(lo, w1e_ref[...], preferred_element_type=jnp.float32)
    h1 = h1 + jnp.dot(hi, w1o_ref[...], preferred_element_type=jnp.float32)
    h1 = jnp.maximum(h1 + b1_ref[...], 0.0)
    o_ref[...] = jnp.dot(h1, w2_ref[...], preferred_element_type=jnp.float32) + b2_ref[...]


@functools.cache
def _mlp_call():
    return pl.pallas_call(
        _mlp_body,
        grid=(N // RT,),
        in_specs=[
            pl.BlockSpec(memory_space=pltpu.SMEM),
            pl.BlockSpec((RT, D), lambda i: (i, 0)),
            pl.BlockSpec((RT, DW), lambda i: (i, 0)),
            pl.BlockSpec((D, D), lambda i: (0, 0)),
            pl.BlockSpec((DW, D), lambda i: (0, 0)),
            pl.BlockSpec((DW, D), lambda i: (0, 0)),
            pl.BlockSpec((1, D), lambda i: (0, 0)),
            pl.BlockSpec((D, D), lambda i: (0, 0)),
            pl.BlockSpec((1, D), lambda i: (0, 0)),
        ],
        out_specs=pl.BlockSpec((RT, D), lambda i: (i, 0)),
        out_shape=jax.ShapeDtypeStruct((N, D), jnp.float32),
    )


def kernel(x, neigh, eps, W1, b1, W2, b2):
    x2d = x[0]
    idx = neigh.astype(jnp.int32)
    idx = jnp.pad(idx, ((0, NP - N), (0, 0)))
    idx = idx.reshape(NW, GROUPS, NBUF * CK)
    y = lax.bitcast_convert_type(x2d, jnp.uint32)
    r = (y + jnp.uint32(0x8000)) >> 16
    xi = lax.bitcast_convert_type(r[:, :DW] | (r[:, DW:] << 16), jnp.int32)
    agg_i = _agg_call()(xi, idx).reshape(NP, DW)
    eps_arr = jnp.reshape(eps, (1,)).astype(jnp.float32)
    out = _mlp_call()(eps_arr, x2d, agg_i, W1, W1[:DW], W1[DW:],
                      jnp.reshape(b1, (1, D)), W2, jnp.reshape(b2, (1, D)))
    return out[None]


# speedup vs baseline: 1.4202x; 1.0080x over previous
import functools

import jax
import jax.numpy as jnp
from jax import lax
from jax.experimental import pallas as pl
from jax.experimental.pallas import tpu as pltpu
from jax.experimental.pallas import tpu_sc as plsc

N = 10000
K = 16
D = 256
LANES = 16
DW = D // 2
GL = DW // LANES
NC = 2
NS = 16
NW = NC * NS
NPW = 320
NP = NW * NPW
C = 2
CK = C * K
CHUNKS = NPW // C
NBUF = 8
GROUPS = CHUNKS // NBUF

_FMT = plsc.PackFormat.INTERLEAVED


def _agg_body(x_hbm, idx_hbm, out_hbm, idx_v, rows_v, agg_v, gsem):
    wid = lax.axis_index("s") * NC + lax.axis_index("c")
    pltpu.sync_copy(idx_hbm.at[wid], idx_v)

    def issue(g, slot, b):
        pltpu.async_copy(
            x_hbm.at[idx_v.at[g, pl.ds(slot * CK, CK)]], rows_v.at[b], gsem)

    def wait(g, slot, b):
        pltpu.make_async_copy(
            x_hbm.at[idx_v.at[g, pl.ds(slot * CK, CK)]], rows_v.at[b], gsem).wait()

    def compute_chunk(c, b):
        def node_body(j, _):
            row0 = j * K
            node = c * C + j
            for t in range(GL):
                col = t * LANES

                def load(k):
                    v = rows_v[b, row0 + k, pl.ds(col, LANES)]
                    return plsc.unpack(plsc.bitcast(v, jnp.bfloat16), format=_FMT)

                sa, sb = load(0)
                for k in range(1, K):
                    pa, pb = load(k)
                    sa = sa + pa
                    sb = sb + pb
                packed = plsc.pack(sa, sb, format=_FMT)
                agg_v[node, pl.ds(col, LANES)] = plsc.bitcast(packed, jnp.int32)
            return 0

        lax.fori_loop(0, C, node_body, 0)

    for b in range(NBUF - 1):
        issue(0, b, b)

    def group_body(i, _):
        c0 = i * NBUF
        for b in range(NBUF):
            c = c0 + b
            wait(i, b, b)
            slot = (b + NBUF - 1) % NBUF
            issue(i if b == 0 else i + 1, slot, slot)
            compute_chunk(c, b)
        return 0

    lax.fori_loop(0, GROUPS - 1, group_body, 0)
    g = GROUPS - 1
    c0 = g * NBUF
    for b in range(NBUF):
        c = c0 + b
        wait(g, b, b)
        if b == 0:
            issue(g, NBUF - 1, NBUF - 1)
        compute_chunk(c, b)
    pltpu.sync_copy(agg_v, out_hbm.at[wid])


@functools.cache
def _agg_call():
    mesh = plsc.VectorSubcoreMesh(core_axis_name="c", subcore_axis_name="s")
    return pl.kernel(
        _agg_body,
        out_type=jax.ShapeDtypeStruct((NW, NPW, DW), jnp.int32),
        mesh=mesh,
        scratch_types=[
            pltpu.VMEM((GROUPS, NBUF * CK), jnp.int32),
            pltpu.VMEM((NBUF, CK, DW), jnp.int32),
            pltpu.VMEM((NPW, DW), jnp.int32),
            pltpu.SemaphoreType.DMA,
        ],
        compiler_params=pltpu.CompilerParams(needs_layout_passes=False),
    )


RT = 1000


def _mlp_body(eps_ref, x_ref, agg_ref, w1_ref, w1e_ref, w1o_ref, b1_ref,
              w2_ref, b2_ref, o_ref):
    w = agg_ref[...]
    lo = lax.bitcast_convert_type(w << 16, jnp.float32)
    hi = lax.bitcast_convert_type(w & jnp.int32(-65536), jnp.float32)
    h1 = (1.0 + eps_ref[0]) * jnp.dot(
        x_ref[...], w1_ref[...], preferred_element_type=jnp.float32)
    h1 = h1 + jnp.dot(lo, w1e_ref[...], preferred_element_type=jnp.float32)
    h1 = h1 + jnp.dot(hi, w1o_ref[...], preferred_element_type=jnp.float32)
    h1 = jnp.maximum(h1 + b1_ref[...], 0.0)
    o_ref[...] = jnp.dot(h1, w2_ref[...], preferred_element_type=jnp.float32) + b2_ref[...]


@functools.cache
def _mlp_call():
    return pl.pallas_call(
        _mlp_body,
        grid=(N // RT,),
        in_specs=[
            pl.BlockSpec(memory_space=pltpu.SMEM),
            pl.BlockSpec((RT, D), lambda i: (i, 0)),
            pl.BlockSpec((RT, DW), lambda i: (i, 0)),
            pl.BlockSpec((D, D), lambda i: (0, 0)),
            pl.BlockSpec((DW, D), lambda i: (0, 0)),
            pl.BlockSpec((DW, D), lambda i: (0, 0)),
            pl.BlockSpec((1, D), lambda i: (0, 0)),
            pl.BlockSpec((D, D), lambda i: (0, 0)),
            pl.BlockSpec((1, D), lambda i: (0, 0)),
        ],
        out_specs=pl.BlockSpec((RT, D), lambda i: (i, 0)),
        out_shape=jax.ShapeDtypeStruct((N, D), jnp.float32),
    )


def kernel(x, neigh, eps, W1, b1, W2, b2):
    x2d = x[0]
    idx = neigh.astype(jnp.int32)
    idx = jnp.pad(idx, ((0, NP - N), (0, 0)))
    idx = idx.reshape(NW, GROUPS, NBUF * CK)
    y = lax.bitcast_convert_type(x2d, jnp.uint32)
    r = (y + jnp.uint32(0x8000)) >> 16
    xi = lax.bitcast_convert_type(r[:, :DW] | (r[:, DW:] << 16), jnp.int32)
    agg_i = _agg_call()(xi, idx).reshape(NP, DW)
    eps_arr = jnp.reshape(eps, (1,)).astype(jnp.float32)
    out = _mlp_call()(eps_arr, x2d, agg_i, W1, W1[:DW], W1[DW:],
                      jnp.reshape(b1, (1, D)), W2, jnp.reshape(b2, (1, D)))
    return out[None]


# speedup vs baseline: 2.8122x; 1.9801x over previous
import functools

import jax
import jax.numpy as jnp
from jax import lax
from jax.experimental import pallas as pl
from jax.experimental.pallas import tpu as pltpu
from jax.experimental.pallas import tpu_sc as plsc

N = 10000
K = 16
D = 256
LANES = 16
DW = D // 2
GL = DW // LANES
NC = 2
NS = 16
NW = NC * NS
NPW = 320
NP = NW * NPW
C = 2
CK = C * K
CHUNKS = NPW // C
NBUF = 8
GROUPS = CHUNKS // NBUF

_FMT = plsc.PackFormat.INTERLEAVED


def _agg_body(x_hbm, idx_hbm, out_hbm, idx_v, rows_v, agg_v, gsem):
    wid = lax.axis_index("s") * NC + lax.axis_index("c")
    pltpu.sync_copy(idx_hbm.at[wid], idx_v)

    def issue(g, slot, b):
        pltpu.async_copy(
            x_hbm.at[idx_v.at[g, pl.ds(slot * CK, CK)]], rows_v.at[b], gsem)

    def wait(g, slot, b):
        pltpu.make_async_copy(
            x_hbm.at[idx_v.at[g, pl.ds(slot * CK, CK)]], rows_v.at[b], gsem).wait()

    def compute_chunk(c, b):
        def node_body(j, _):
            row0 = j * K
            node = c * C + j
            for t in range(GL):
                col = t * LANES

                def load(k):
                    v = rows_v[b, row0 + k, pl.ds(col, LANES)]
                    return plsc.unpack(plsc.bitcast(v, jnp.bfloat16), format=_FMT)

                sa, sb = load(0)
                for k in range(1, K):
                    pa, pb = load(k)
                    sa = sa + pa
                    sb = sb + pb
                packed = plsc.pack(sa, sb, format=_FMT)
                agg_v[node, pl.ds(col, LANES)] = plsc.bitcast(packed, jnp.int32)
            return 0

        lax.fori_loop(0, C, node_body, 0)

    for b in range(NBUF - 1):
        issue(0, b, b)

    def group_body(i, _):
        c0 = i * NBUF
        for b in range(NBUF):
            c = c0 + b
            wait(i, b, b)
            slot = (b + NBUF - 1) % NBUF
            issue(i if b == 0 else i + 1, slot, slot)
            compute_chunk(c, b)
        return 0

    lax.fori_loop(0, GROUPS - 1, group_body, 0)
    g = GROUPS - 1
    c0 = g * NBUF
    for b in range(NBUF):
        c = c0 + b
        wait(g, b, b)
        if b == 0:
            issue(g, NBUF - 1, NBUF - 1)
        compute_chunk(c, b)
    pltpu.sync_copy(agg_v, out_hbm.at[wid])


@functools.cache
def _agg_call():
    mesh = plsc.VectorSubcoreMesh(core_axis_name="c", subcore_axis_name="s")
    return pl.kernel(
        _agg_body,
        out_type=jax.ShapeDtypeStruct((NW, NPW, DW), jnp.int32),
        mesh=mesh,
        scratch_types=[
            pltpu.VMEM((GROUPS, NBUF * CK), jnp.int32),
            pltpu.VMEM((NBUF, CK, DW), jnp.int32),
            pltpu.VMEM((NPW, DW), jnp.int32),
            pltpu.SemaphoreType.DMA,
        ],
        compiler_params=pltpu.CompilerParams(needs_layout_passes=False),
    )


RT = 1000


def _mlp_body(eps_ref, x_ref, agg_ref, w1_ref, w1e_ref, w1o_ref, b1_ref,
              w2_ref, b2_ref, o_ref):
    w = agg_ref[...]
    lo = lax.bitcast_convert_type(w << 16, jnp.float32)
    hi = lax.bitcast_convert_type(w & jnp.int32(-65536), jnp.float32)
    h1 = (1.0 + eps_ref[0]) * jnp.dot(
        x_ref[...], w1_ref[...], preferred_element_type=jnp.float32)
    h1 = h1 + jnp.dot(lo, w1e_ref[...], preferred_element_type=jnp.float32)
    h1 = h1 + jnp.dot(hi, w1o_ref[...], preferred_element_type=jnp.float32)
    h1 = jnp.maximum(h1 + b1_ref[...], 0.0)
    o_ref[...] = jnp.dot(h1, w2_ref[...], preferred_element_type=jnp.float32) + b2_ref[...]


@functools.cache
def _mlp_call():
    return pl.pallas_call(
        _mlp_body,
        grid=(N // RT,),
        in_specs=[
            pl.BlockSpec(memory_space=pltpu.SMEM),
            pl.BlockSpec((RT, D), lambda i: (i, 0)),
            pl.BlockSpec((RT, DW), lambda i: (i, 0)),
            pl.BlockSpec((D, D), lambda i: (0, 0)),
            pl.BlockSpec((DW, D), lambda i: (0, 0)),
            pl.BlockSpec((DW, D), lambda i: (0, 0)),
            pl.BlockSpec((1, D), lambda i: (0, 0)),
            pl.BlockSpec((D, D), lambda i: (0, 0)),
            pl.BlockSpec((1, D), lambda i: (0, 0)),
        ],
        out_specs=pl.BlockSpec((RT, D), lambda i: (i, 0)),
        out_shape=jax.ShapeDtypeStruct((N, D), jnp.float32),
    )


def kernel(x, neigh, eps, W1, b1, W2, b2):
    x2d = x[0]
    idx = neigh.astype(jnp.int32)
    pad_idx = (jnp.arange((NP - N) * K, dtype=jnp.int32) % N).reshape(NP - N, K)
    idx = jnp.concatenate([idx, pad_idx], axis=0)
    idx = idx.reshape(NW, GROUPS, NBUF * CK)
    y = lax.bitcast_convert_type(x2d, jnp.uint32)
    r = (y + jnp.uint32(0x8000)) >> 16
    xi = lax.bitcast_convert_type(r[:, :DW] | (r[:, DW:] << 16), jnp.int32)
    agg_i = _agg_call()(xi, idx).reshape(NP, DW)
    eps_arr = jnp.reshape(eps, (1,)).astype(jnp.float32)
    out = _mlp_call()(eps_arr, x2d, agg_i, W1, W1[:DW], W1[DW:],
                      jnp.reshape(b1, (1, D)), W2, jnp.reshape(b2, (1, D)))
    return out[None]


# speedup vs baseline: 3.3471x; 1.1902x over previous
import functools

import jax
import jax.numpy as jnp
from jax import lax
from jax.experimental import pallas as pl
from jax.experimental.pallas import tpu as pltpu
from jax.experimental.pallas import tpu_sc as plsc

N = 10000
K = 16
D = 256
LANES = 16
DW = D // 2
GL = DW // LANES
NC = 2
NS = 16
NW = NC * NS
NPW = 320
NP = NW * NPW
C = 4
CK = C * K
CHUNKS = NPW // C
NBUF = 8
GROUPS = CHUNKS // NBUF

_FMT = plsc.PackFormat.INTERLEAVED


def _agg_body(x_hbm, idx_hbm, out_hbm, idx_v, rows_v, agg_v, gsem):
    wid = lax.axis_index("s") * NC + lax.axis_index("c")
    pltpu.sync_copy(idx_hbm.at[wid], idx_v)

    def issue(g, slot, b):
        pltpu.async_copy(
            x_hbm.at[idx_v.at[g, pl.ds(slot * CK, CK)]], rows_v.at[b], gsem)

    def wait(g, slot, b):
        pltpu.make_async_copy(
            x_hbm.at[idx_v.at[g, pl.ds(slot * CK, CK)]], rows_v.at[b], gsem).wait()

    def compute_chunk(c, b):
        def node_body(j, _):
            row0 = j * K
            node = c * C + j
            for t in range(GL):
                col = t * LANES

                def load(k):
                    v = rows_v[b, row0 + k, pl.ds(col, LANES)]
                    return plsc.unpack(plsc.bitcast(v, jnp.bfloat16), format=_FMT)

                sa, sb = load(0)
                for k in range(1, K):
                    pa, pb = load(k)
                    sa = sa + pa
                    sb = sb + pb
                packed = plsc.pack(sa, sb, format=_FMT)
                agg_v[node, pl.ds(col, LANES)] = plsc.bitcast(packed, jnp.int32)
            return 0

        lax.fori_loop(0, C, node_body, 0)

    for b in range(NBUF - 1):
        issue(0, b, b)

    def group_body(i, _):
        c0 = i * NBUF
        for b in range(NBUF):
            c = c0 + b
            wait(i, b, b)
            slot = (b + NBUF - 1) % NBUF
            issue(i if b == 0 else i + 1, slot, slot)
            compute_chunk(c, b)
        return 0

    lax.fori_loop(0, GROUPS - 1, group_body, 0)
    g = GROUPS - 1
    c0 = g * NBUF
    for b in range(NBUF):
        c = c0 + b
        wait(g, b, b)
        if b == 0:
            issue(g, NBUF - 1, NBUF - 1)
        compute_chunk(c, b)
    pltpu.sync_copy(agg_v, out_hbm.at[wid])


@functools.cache
def _agg_call():
    mesh = plsc.VectorSubcoreMesh(core_axis_name="c", subcore_axis_name="s")
    return pl.kernel(
        _agg_body,
        out_type=jax.ShapeDtypeStruct((NW, NPW, DW), jnp.int32),
        mesh=mesh,
        scratch_types=[
            pltpu.VMEM((GROUPS, NBUF * CK), jnp.int32),
            pltpu.VMEM((NBUF, CK, DW), jnp.int32),
            pltpu.VMEM((NPW, DW), jnp.int32),
            pltpu.SemaphoreType.DMA,
        ],
        compiler_params=pltpu.CompilerParams(needs_layout_passes=False),
    )


RT = 1000


def _mlp_body(eps_ref, x_ref, agg_ref, w1_ref, w1e_ref, w1o_ref, b1_ref,
              w2_ref, b2_ref, o_ref):
    w = agg_ref[...]
    lo = lax.bitcast_convert_type(w << 16, jnp.float32)
    hi = lax.bitcast_convert_type(w & jnp.int32(-65536), jnp.float32)
    h1 = (1.0 + eps_ref[0]) * jnp.dot(
        x_ref[...], w1_ref[...], preferred_element_type=jnp.float32)
    h1 = h1 + jnp.dot(lo, w1e_ref[...], preferred_element_type=jnp.float32)
    h1 = h1 + jnp.dot(hi, w1o_ref[...], preferred_element_type=jnp.float32)
    h1 = jnp.maximum(h1 + b1_ref[...], 0.0)
    o_ref[...] = jnp.dot(h1, w2_ref[...], preferred_element_type=jnp.float32) + b2_ref[...]


@functools.cache
def _mlp_call():
    return pl.pallas_call(
        _mlp_body,
        grid=(N // RT,),
        in_specs=[
            pl.BlockSpec(memory_space=pltpu.SMEM),
            pl.BlockSpec((RT, D), lambda i: (i, 0)),
            pl.BlockSpec((RT, DW), lambda i: (i, 0)),
            pl.BlockSpec((D, D), lambda i: (0, 0)),
            pl.BlockSpec((DW, D), lambda i: (0, 0)),
            pl.BlockSpec((DW, D), lambda i: (0, 0)),
            pl.BlockSpec((1, D), lambda i: (0, 0)),
            pl.BlockSpec((D, D), lambda i: (0, 0)),
            pl.BlockSpec((1, D), lambda i: (0, 0)),
        ],
        out_specs=pl.BlockSpec((RT, D), lambda i: (i, 0)),
        out_shape=jax.ShapeDtypeStruct((N, D), jnp.float32),
    )


def kernel(x, neigh, eps, W1, b1, W2, b2):
    x2d = x[0]
    idx = neigh.astype(jnp.int32)
    pad_idx = (jnp.arange((NP - N) * K, dtype=jnp.int32) % N).reshape(NP - N, K)
    idx = jnp.concatenate([idx, pad_idx], axis=0)
    idx = idx.reshape(NW, GROUPS, NBUF * CK)
    y = lax.bitcast_convert_type(x2d, jnp.uint32)
    r = (y + jnp.uint32(0x8000)) >> 16
    xi = lax.bitcast_convert_type(r[:, :DW] | (r[:, DW:] << 16), jnp.int32)
    agg_i = _agg_call()(xi, idx).reshape(NP, DW)
    eps_arr = jnp.reshape(eps, (1,)).astype(jnp.float32)
    out = _mlp_call()(eps_arr, x2d, agg_i, W1, W1[:DW], W1[DW:],
                      jnp.reshape(b1, (1, D)), W2, jnp.reshape(b2, (1, D)))
    return out[None]


# speedup vs baseline: 3.6051x; 1.0771x over previous
import functools

import jax
import jax.numpy as jnp
from jax import lax
from jax.experimental import pallas as pl
from jax.experimental.pallas import tpu as pltpu
from jax.experimental.pallas import tpu_sc as plsc

N = 10000
K = 16
D = 256
LANES = 16
DW = D // 2
GL = DW // LANES
NC = 2
NS = 16
NW = NC * NS
NPW = 320
NP = NW * NPW
C = 8
CK = C * K
CHUNKS = NPW // C
NBUF = 4
GROUPS = CHUNKS // NBUF

_FMT = plsc.PackFormat.INTERLEAVED


def _agg_body(x_hbm, idx_hbm, out_hbm, idx_v, rows_v, agg_v, gsem):
    wid = lax.axis_index("s") * NC + lax.axis_index("c")
    pltpu.sync_copy(idx_hbm.at[wid], idx_v)

    def issue(g, slot, b):
        pltpu.async_copy(
            x_hbm.at[idx_v.at[g, pl.ds(slot * CK, CK)]], rows_v.at[b], gsem)

    def wait(g, slot, b):
        pltpu.make_async_copy(
            x_hbm.at[idx_v.at[g, pl.ds(slot * CK, CK)]], rows_v.at[b], gsem).wait()

    def compute_chunk(c, b):
        def node_body(j, _):
            row0 = j * K
            node = c * C + j
            for t in range(GL):
                col = t * LANES

                def load(k):
                    v = rows_v[b, row0 + k, pl.ds(col, LANES)]
                    return plsc.unpack(plsc.bitcast(v, jnp.bfloat16), format=_FMT)

                sa, sb = load(0)
                for k in range(1, K):
                    pa, pb = load(k)
                    sa = sa + pa
                    sb = sb + pb
                packed = plsc.pack(sa, sb, format=_FMT)
                agg_v[node, pl.ds(col, LANES)] = plsc.bitcast(packed, jnp.int32)
            return 0

        lax.fori_loop(0, C, node_body, 0)

    for b in range(NBUF - 1):
        issue(0, b, b)

    def group_body(i, _):
        c0 = i * NBUF
        for b in range(NBUF):
            c = c0 + b
            wait(i, b, b)
            slot = (b + NBUF - 1) % NBUF
            issue(i if b == 0 else i + 1, slot, slot)
            compute_chunk(c, b)
        return 0

    lax.fori_loop(0, GROUPS - 1, group_body, 0)
    g = GROUPS - 1
    c0 = g * NBUF
    for b in range(NBUF):
        c = c0 + b
        wait(g, b, b)
        if b == 0:
            issue(g, NBUF - 1, NBUF - 1)
        compute_chunk(c, b)
    pltpu.sync_copy(agg_v, out_hbm.at[wid])


@functools.cache
def _agg_call():
    mesh = plsc.VectorSubcoreMesh(core_axis_name="c", subcore_axis_name="s")
    return pl.kernel(
        _agg_body,
        out_type=jax.ShapeDtypeStruct((NW, NPW, DW), jnp.int32),
        mesh=mesh,
        scratch_types=[
            pltpu.VMEM((GROUPS, NBUF * CK), jnp.int32),
            pltpu.VMEM((NBUF, CK, DW), jnp.int32),
            pltpu.VMEM((NPW, DW), jnp.int32),
            pltpu.SemaphoreType.DMA,
        ],
        compiler_params=pltpu.CompilerParams(needs_layout_passes=False),
    )


RT = 1000


def _mlp_body(eps_ref, x_ref, agg_ref, w1_ref, w1e_ref, w1o_ref, b1_ref,
              w2_ref, b2_ref, o_ref):
    w = agg_ref[...]
    lo = lax.bitcast_convert_type(w << 16, jnp.float32)
    hi = lax.bitcast_convert_type(w & jnp.int32(-65536), jnp.float32)
    h1 = (1.0 + eps_ref[0]) * jnp.dot(
        x_ref[...], w1_ref[...], preferred_element_type=jnp.float32)
    h1 = h1 + jnp.dot(lo, w1e_ref[...], preferred_element_type=jnp.float32)
    h1 = h1 + jnp.dot(hi, w1o_ref[...], preferred_element_type=jnp.float32)
    h1 = jnp.maximum(h1 + b1_ref[...], 0.0)
    o_ref[...] = jnp.dot(h1, w2_ref[...], preferred_element_type=jnp.float32) + b2_ref[...]


@functools.cache
def _mlp_call():
    return pl.pallas_call(
        _mlp_body,
        grid=(N // RT,),
        in_specs=[
            pl.BlockSpec(memory_space=pltpu.SMEM),
            pl.BlockSpec((RT, D), lambda i: (i, 0)),
            pl.BlockSpec((RT, DW), lambda i: (i, 0)),
            pl.BlockSpec((D, D), lambda i: (0, 0)),
            pl.BlockSpec((DW, D), lambda i: (0, 0)),
            pl.BlockSpec((DW, D), lambda i: (0, 0)),
            pl.BlockSpec((1, D), lambda i: (0, 0)),
            pl.BlockSpec((D, D), lambda i: (0, 0)),
            pl.BlockSpec((1, D), lambda i: (0, 0)),
        ],
        out_specs=pl.BlockSpec((RT, D), lambda i: (i, 0)),
        out_shape=jax.ShapeDtypeStruct((N, D), jnp.float32),
    )


def kernel(x, neigh, eps, W1, b1, W2, b2):
    x2d = x[0]
    idx = neigh.astype(jnp.int32)
    pad_idx = (jnp.arange((NP - N) * K, dtype=jnp.int32) % N).reshape(NP - N, K)
    idx = jnp.concatenate([idx, pad_idx], axis=0)
    idx = idx.reshape(NW, GROUPS, NBUF * CK)
    y = lax.bitcast_convert_type(x2d, jnp.uint32)
    r = (y + jnp.uint32(0x8000)) >> 16
    xi = lax.bitcast_convert_type(r[:, :DW] | (r[:, DW:] << 16), jnp.int32)
    agg_i = _agg_call()(xi, idx).reshape(NP, DW)
    eps_arr = jnp.reshape(eps, (1,)).astype(jnp.float32)
    out = _mlp_call()(eps_arr, x2d, agg_i, W1, W1[:DW], W1[DW:],
                      jnp.reshape(b1, (1, D)), W2, jnp.reshape(b2, (1, D)))
    return out[None]


# speedup vs baseline: 4.2494x; 1.1787x over previous
import functools

import jax
import jax.numpy as jnp
from jax import lax
from jax.experimental import pallas as pl
from jax.experimental.pallas import tpu as pltpu
from jax.experimental.pallas import tpu_sc as plsc

N = 10000
K = 16
D = 256
LANES = 16
DW = D // 2
GL = DW // LANES
NC = 2
NS = 16
NW = NC * NS
NPW = 320
NP = NW * NPW
C = 8
CK = C * K
CHUNKS = NPW // C
NBUF = 4
GROUPS = CHUNKS // NBUF

_FMT = plsc.PackFormat.INTERLEAVED


def _agg_body(x_hbm, idx_hbm, out_hbm, idx_v, rows_v, agg_v, gsem):
    wid = lax.axis_index("s") * NC + lax.axis_index("c")
    pltpu.sync_copy(idx_hbm.at[wid], idx_v)

    def issue(g, slot, b):
        pltpu.async_copy(
            x_hbm.at[idx_v.at[g, pl.ds(slot * CK, CK)]], rows_v.at[b], gsem)

    def wait(g, slot, b):
        pltpu.make_async_copy(
            x_hbm.at[idx_v.at[g, pl.ds(slot * CK, CK)]], rows_v.at[b], gsem).wait()

    def compute_chunk(c, b):
        def node_body(j, _):
            row0 = j * K
            node = c * C + j
            for t in range(GL):
                col = t * LANES

                def load(k):
                    v = rows_v[b, row0 + k, pl.ds(col, LANES)]
                    return plsc.bitcast(v, jnp.bfloat16)

                sa, sb = plsc.unpack(load(0) + load(1), format=_FMT)
                for k in range(2, K, 2):
                    pa, pb = plsc.unpack(load(k) + load(k + 1), format=_FMT)
                    sa = sa + pa
                    sb = sb + pb
                packed = plsc.pack(sa, sb, format=_FMT)
                agg_v[node, pl.ds(col, LANES)] = plsc.bitcast(packed, jnp.int32)
            return 0

        lax.fori_loop(0, C, node_body, 0)

    for b in range(NBUF - 1):
        issue(0, b, b)

    def group_body(i, _):
        c0 = i * NBUF
        for b in range(NBUF):
            c = c0 + b
            wait(i, b, b)
            slot = (b + NBUF - 1) % NBUF
            issue(i if b == 0 else i + 1, slot, slot)
            compute_chunk(c, b)
        return 0

    lax.fori_loop(0, GROUPS - 1, group_body, 0)
    g = GROUPS - 1
    c0 = g * NBUF
    for b in range(NBUF):
        c = c0 + b
        wait(g, b, b)
        if b == 0:
            issue(g, NBUF - 1, NBUF - 1)
        compute_chunk(c, b)
    pltpu.sync_copy(agg_v, out_hbm.at[wid])


@functools.cache
def _agg_call():
    mesh = plsc.VectorSubcoreMesh(core_axis_name="c", subcore_axis_name="s")
    return pl.kernel(
        _agg_body,
        out_type=jax.ShapeDtypeStruct((NW, NPW, DW), jnp.int32),
        mesh=mesh,
        scratch_types=[
            pltpu.VMEM((GROUPS, NBUF * CK), jnp.int32),
            pltpu.VMEM((NBUF, CK, DW), jnp.int32),
            pltpu.VMEM((NPW, DW), jnp.int32),
            pltpu.SemaphoreType.DMA,
        ],
        compiler_params=pltpu.CompilerParams(needs_layout_passes=False),
    )


RT = 1000


def _mlp_body(eps_ref, x_ref, agg_ref, w1_ref, w1e_ref, w1o_ref, b1_ref,
              w2_ref, b2_ref, o_ref):
    w = agg_ref[...]
    lo = lax.bitcast_convert_type(w << 16, jnp.float32)
    hi = lax.bitcast_convert_type(w & jnp.int32(-65536), jnp.float32)
    h1 = (1.0 + eps_ref[0]) * jnp.dot(
        x_ref[...], w1_ref[...], preferred_element_type=jnp.float32)
    h1 = h1 + jnp.dot(lo, w1e_ref[...], preferred_element_type=jnp.float32)
    h1 = h1 + jnp.dot(hi, w1o_ref[...], preferred_element_type=jnp.float32)
    h1 = jnp.maximum(h1 + b1_ref[...], 0.0)
    o_ref[...] = jnp.dot(h1, w2_ref[...], preferred_element_type=jnp.float32) + b2_ref[...]


@functools.cache
def _mlp_call():
    return pl.pallas_call(
        _mlp_body,
        grid=(N // RT,),
        in_specs=[
            pl.BlockSpec(memory_space=pltpu.SMEM),
            pl.BlockSpec((RT, D), lambda i: (i, 0)),
            pl.BlockSpec((RT, DW), lambda i: (i, 0)),
            pl.BlockSpec((D, D), lambda i: (0, 0)),
            pl.BlockSpec((DW, D), lambda i: (0, 0)),
            pl.BlockSpec((DW, D), lambda i: (0, 0)),
            pl.BlockSpec((1, D), lambda i: (0, 0)),
            pl.BlockSpec((D, D), lambda i: (0, 0)),
            pl.BlockSpec((1, D), lambda i: (0, 0)),
        ],
        out_specs=pl.BlockSpec((RT, D), lambda i: (i, 0)),
        out_shape=jax.ShapeDtypeStruct((N, D), jnp.float32),
    )


def kernel(x, neigh, eps, W1, b1, W2, b2):
    x2d = x[0]
    idx = neigh.astype(jnp.int32)
    pad_idx = (jnp.arange((NP - N) * K, dtype=jnp.int32) % N).reshape(NP - N, K)
    idx = jnp.concatenate([idx, pad_idx], axis=0)
    idx = idx.reshape(NW, GROUPS, NBUF * CK)
    y = lax.bitcast_convert_type(x2d, jnp.uint32)
    r = (y + jnp.uint32(0x8000)) >> 16
    xi = lax.bitcast_convert_type(r[:, :DW] | (r[:, DW:] << 16), jnp.int32)
    agg_i = _agg_call()(xi, idx).reshape(NP, DW)
    eps_arr = jnp.reshape(eps, (1,)).astype(jnp.float32)
    out = _mlp_call()(eps_arr, x2d, agg_i, W1, W1[:DW], W1[DW:],
                      jnp.reshape(b1, (1, D)), W2, jnp.reshape(b2, (1, D)))
    return out[None]


# speedup vs baseline: 4.2712x; 1.0051x over previous
import functools

import jax
import jax.numpy as jnp
from jax import lax
from jax.experimental import pallas as pl
from jax.experimental.pallas import tpu as pltpu
from jax.experimental.pallas import tpu_sc as plsc

N = 10000
K = 16
D = 256
LANES = 16
DW = D // 2
GL = DW // LANES
NC = 2
NS = 16
NW = NC * NS
NPW = 320
NP = NW * NPW
C = 8
CK = C * K
CHUNKS = NPW // C
NBUF = 4
GROUPS = CHUNKS // NBUF

_FMT = plsc.PackFormat.INTERLEAVED


def _agg_body(x_hbm, idx_hbm, out_hbm, idx_v, rows_v, agg_v, gsem):
    wid = lax.axis_index("s") * NC + lax.axis_index("c")
    pltpu.sync_copy(idx_hbm.at[wid], idx_v)

    def issue(g, slot, b):
        pltpu.async_copy(
            x_hbm.at[idx_v.at[g, pl.ds(slot * CK, CK)]], rows_v.at[b], gsem)

    def wait(g, slot, b):
        pltpu.make_async_copy(
            x_hbm.at[idx_v.at[g, pl.ds(slot * CK, CK)]], rows_v.at[b], gsem).wait()

    def compute_chunk(c, b):
        def node_body(j, _):
            row0 = j * K
            node = c * C + j
            for t in range(GL):
                col = t * LANES

                def load(k):
                    v = rows_v[b, row0 + k, pl.ds(col, LANES)]
                    return plsc.bitcast(v, jnp.bfloat16)

                sa, sb = plsc.unpack(load(0) + load(1), format=_FMT)
                for k in range(2, K, 2):
                    pa, pb = plsc.unpack(load(k) + load(k + 1), format=_FMT)
                    sa = sa + pa
                    sb = sb + pb
                packed = plsc.pack(sa, sb, format=_FMT)
                agg_v[node, pl.ds(col, LANES)] = plsc.bitcast(packed, jnp.int32)
            return 0

        lax.fori_loop(0, C, node_body, 0)

    for b in range(NBUF - 1):
        issue(0, b, b)

    def group_body(i, _):
        c0 = i * NBUF
        for b in range(NBUF):
            c = c0 + b
            wait(i, b, b)
            slot = (b + NBUF - 1) % NBUF
            issue(i if b == 0 else i + 1, slot, slot)
            compute_chunk(c, b)
        return 0

    lax.fori_loop(0, GROUPS - 1, group_body, 0)
    g = GROUPS - 1
    c0 = g * NBUF
    for b in range(NBUF):
        c = c0 + b
        wait(g, b, b)
        if b == 0:
            issue(g, NBUF - 1, NBUF - 1)
        compute_chunk(c, b)
    pltpu.sync_copy(agg_v, out_hbm.at[wid])


@functools.cache
def _agg_call():
    mesh = plsc.VectorSubcoreMesh(core_axis_name="c", subcore_axis_name="s")
    return pl.kernel(
        _agg_body,
        out_type=jax.ShapeDtypeStruct((NW, NPW, DW), jnp.int32),
        mesh=mesh,
        scratch_types=[
            pltpu.VMEM((GROUPS, NBUF * CK), jnp.int32),
            pltpu.VMEM((NBUF, CK, DW), jnp.int32),
            pltpu.VMEM((NPW, DW), jnp.int32),
            pltpu.SemaphoreType.DMA,
        ],
        compiler_params=pltpu.CompilerParams(needs_layout_passes=False),
    )


RT = 1000


def _mlp_body(eps_ref, xi_ref, agg_ref, w1e_ref, w1o_ref, b1_ref,
              w2_ref, b2_ref, o_ref):
    def halves(w):
        return (lax.bitcast_convert_type(w << 16, jnp.float32),
                lax.bitcast_convert_type(w & jnp.int32(-65536), jnp.float32))

    s = 1.0 + eps_ref[0]
    lx, hx = halves(xi_ref[...])
    la, ha = halves(agg_ref[...])
    lc = s * lx + la
    hc = s * hx + ha
    h1 = jnp.dot(lc, w1e_ref[...], preferred_element_type=jnp.float32)
    h1 = h1 + jnp.dot(hc, w1o_ref[...], preferred_element_type=jnp.float32)
    h1 = jnp.maximum(h1 + b1_ref[...], 0.0)
    o_ref[...] = jnp.dot(h1, w2_ref[...], preferred_element_type=jnp.float32) + b2_ref[...]


@functools.cache
def _mlp_call():
    return pl.pallas_call(
        _mlp_body,
        grid=(N // RT,),
        in_specs=[
            pl.BlockSpec(memory_space=pltpu.SMEM),
            pl.BlockSpec((RT, DW), lambda i: (i, 0)),
            pl.BlockSpec((RT, DW), lambda i: (i, 0)),
            pl.BlockSpec((DW, D), lambda i: (0, 0)),
            pl.BlockSpec((DW, D), lambda i: (0, 0)),
            pl.BlockSpec((1, D), lambda i: (0, 0)),
            pl.BlockSpec((D, D), lambda i: (0, 0)),
            pl.BlockSpec((1, D), lambda i: (0, 0)),
        ],
        out_specs=pl.BlockSpec((RT, D), lambda i: (i, 0)),
        out_shape=jax.ShapeDtypeStruct((N, D), jnp.float32),
    )


def kernel(x, neigh, eps, W1, b1, W2, b2):
    x2d = x[0]
    idx = neigh.astype(jnp.int32)
    pad_idx = (jnp.arange((NP - N) * K, dtype=jnp.int32) % N).reshape(NP - N, K)
    idx = jnp.concatenate([idx, pad_idx], axis=0)
    idx = idx.reshape(NW, GROUPS, NBUF * CK)
    y = lax.bitcast_convert_type(x2d, jnp.uint32)
    r = (y + jnp.uint32(0x8000)) >> 16
    xi = lax.bitcast_convert_type(r[:, :DW] | (r[:, DW:] << 16), jnp.int32)
    agg_i = _agg_call()(xi, idx).reshape(NP, DW)
    eps_arr = jnp.reshape(eps, (1,)).astype(jnp.float32)
    out = _mlp_call()(eps_arr, xi, agg_i, W1[:DW], W1[DW:],
                      jnp.reshape(b1, (1, D)), W2, jnp.reshape(b2, (1, D)))
    return out[None]


# speedup vs baseline: 4.3877x; 1.0273x over previous
import functools

import jax
import jax.numpy as jnp
from jax import lax
from jax.experimental import pallas as pl
from jax.experimental.pallas import tpu as pltpu
from jax.experimental.pallas import tpu_sc as plsc

N = 10000
K = 16
D = 256
LANES = 16
DW = D // 2
GL = DW // LANES
NC = 2
NS = 16
NW = NC * NS
NPW = 320
NP = NW * NPW
C = 8
CK = C * K
CHUNKS = NPW // C
NBUF = 4
GROUPS = CHUNKS // NBUF

_FMT = plsc.PackFormat.INTERLEAVED


def _agg_body(x_hbm, idx_hbm, out_hbm, idx_v, rows_v, agg_v, gsem):
    wid = lax.axis_index("s") * NC + lax.axis_index("c")
    pltpu.sync_copy(idx_hbm.at[wid], idx_v)

    def issue(g, slot, b):
        pltpu.async_copy(
            x_hbm.at[idx_v.at[g, pl.ds(slot * CK, CK)]], rows_v.at[b], gsem)

    def wait(g, slot, b):
        pltpu.make_async_copy(
            x_hbm.at[idx_v.at[g, pl.ds(slot * CK, CK)]], rows_v.at[b], gsem).wait()

    def compute_chunk(c, b):
        def node_body(j, _):
            row0 = j * K
            node = c * C + j
            for t in range(GL):
                col = t * LANES

                def load(k):
                    v = rows_v[b, row0 + k, pl.ds(col, LANES)]
                    return plsc.bitcast(v, jnp.bfloat16)

                sa, sb = plsc.unpack(load(0) + load(1), format=_FMT)
                for k in range(2, K, 2):
                    pa, pb = plsc.unpack(load(k) + load(k + 1), format=_FMT)
                    sa = sa + pa
                    sb = sb + pb
                packed = plsc.pack(sa, sb, format=_FMT)
                agg_v[node, pl.ds(col, LANES)] = plsc.bitcast(packed, jnp.int32)
            return 0

        lax.fori_loop(0, C, node_body, 0)

    for b in range(NBUF - 1):
        issue(0, b, b)

    def group_body(i, _):
        c0 = i * NBUF
        for b in range(NBUF):
            c = c0 + b
            wait(i, b, b)
            slot = (b + NBUF - 1) % NBUF
            issue(i if b == 0 else i + 1, slot, slot)
            compute_chunk(c, b)
        return 0

    lax.fori_loop(0, GROUPS - 1, group_body, 0)
    g = GROUPS - 1
    c0 = g * NBUF
    for b in range(NBUF):
        c = c0 + b
        wait(g, b, b)
        if b == 0:
            issue(g, NBUF - 1, NBUF - 1)
        compute_chunk(c, b)
    pltpu.sync_copy(agg_v, out_hbm.at[wid])


@functools.cache
def _agg_call():
    mesh = plsc.VectorSubcoreMesh(core_axis_name="c", subcore_axis_name="s")
    return pl.kernel(
        _agg_body,
        out_type=jax.ShapeDtypeStruct((NW, NPW, DW), jnp.int32),
        mesh=mesh,
        scratch_types=[
            pltpu.VMEM((GROUPS, NBUF * CK), jnp.int32),
            pltpu.VMEM((NBUF, CK, DW), jnp.int32),
            pltpu.VMEM((NPW, DW), jnp.int32),
            pltpu.SemaphoreType.DMA,
        ],
        compiler_params=pltpu.CompilerParams(needs_layout_passes=False, skip_device_barrier=True),
    )


RT = 2000


def _mlp_body(eps_ref, xi_ref, agg_ref, w1e_ref, w1o_ref, b1_ref,
              w2_ref, b2_ref, o_ref):
    def halves(w):
        return (lax.bitcast_convert_type(w << 16, jnp.float32),
                lax.bitcast_convert_type(w & jnp.int32(-65536), jnp.float32))

    s = 1.0 + eps_ref[0]
    lx, hx = halves(xi_ref[...])
    la, ha = halves(agg_ref[...])
    lc = s * lx + la
    hc = s * hx + ha
    h1 = jnp.dot(lc, w1e_ref[...], preferred_element_type=jnp.float32)
    h1 = h1 + jnp.dot(hc, w1o_ref[...], preferred_element_type=jnp.float32)
    h1 = jnp.maximum(h1 + b1_ref[...], 0.0)
    o_ref[...] = jnp.dot(h1, w2_ref[...], preferred_element_type=jnp.float32) + b2_ref[...]


@functools.cache
def _mlp_call():
    return pl.pallas_call(
        _mlp_body,
        grid=(N // RT,),
        in_specs=[
            pl.BlockSpec(memory_space=pltpu.SMEM),
            pl.BlockSpec((RT, DW), lambda i: (i, 0)),
            pl.BlockSpec((RT, DW), lambda i: (i, 0)),
            pl.BlockSpec((DW, D), lambda i: (0, 0)),
            pl.BlockSpec((DW, D), lambda i: (0, 0)),
            pl.BlockSpec((1, D), lambda i: (0, 0)),
            pl.BlockSpec((D, D), lambda i: (0, 0)),
            pl.BlockSpec((1, D), lambda i: (0, 0)),
        ],
        out_specs=pl.BlockSpec((RT, D), lambda i: (i, 0)),
        out_shape=jax.ShapeDtypeStruct((N, D), jnp.float32),
    )


def kernel(x, neigh, eps, W1, b1, W2, b2):
    x2d = x[0]
    idx = neigh.astype(jnp.int32)
    pad_idx = (jnp.arange((NP - N) * K, dtype=jnp.int32) % N).reshape(NP - N, K)
    idx = jnp.concatenate([idx, pad_idx], axis=0)
    idx = idx.reshape(NW, GROUPS, NBUF * CK)
    y = lax.bitcast_convert_type(x2d, jnp.uint32)
    r = (y + jnp.uint32(0x8000)) >> 16
    xi = lax.bitcast_convert_type(r[:, :DW] | (r[:, DW:] << 16), jnp.int32)
    agg_i = _agg_call()(xi, idx).reshape(NP, DW)
    eps_arr = jnp.reshape(eps, (1,)).astype(jnp.float32)
    out = _mlp_call()(eps_arr, xi, agg_i, W1[:DW], W1[DW:],
                      jnp.reshape(b1, (1, D)), W2, jnp.reshape(b2, (1, D)))
    return out[None]


# speedup vs baseline: 4.3896x; 1.0004x over previous
import functools

import jax
import jax.numpy as jnp
from jax import lax
from jax.experimental import pallas as pl
from jax.experimental.pallas import tpu as pltpu
from jax.experimental.pallas import tpu_sc as plsc

N = 10000
K = 16
D = 256
LANES = 16
DW = D // 2
GL = DW // LANES
NC = 2
NS = 16
NW = NC * NS
NPW = 320
NP = NW * NPW
C = 8
CK = C * K
CHUNKS = NPW // C
NBUF = 4
GROUPS = CHUNKS // NBUF

_FMT = plsc.PackFormat.INTERLEAVED


def _agg_body(x_hbm, idx_hbm, out_hbm, idx_v, rows_v, agg_v, gsem):
    wid = lax.axis_index("s") * NC + lax.axis_index("c")
    pltpu.sync_copy(idx_hbm.at[wid], idx_v)

    def issue(g, slot, b):
        pltpu.async_copy(
            x_hbm.at[idx_v.at[g, pl.ds(slot * CK, CK)]], rows_v.at[b], gsem)

    def wait(g, slot, b):
        pltpu.make_async_copy(
            x_hbm.at[idx_v.at[g, pl.ds(slot * CK, CK)]], rows_v.at[b], gsem).wait()

    def compute_chunk(c, b):
        def node_body(j, _):
            row0 = j * K
            node = c * C + j
            for t in range(GL):
                col = t * LANES

                def load(k):
                    v = rows_v[b, row0 + k, pl.ds(col, LANES)]
                    return plsc.bitcast(v, jnp.bfloat16)

                sa, sb = plsc.unpack(load(0) + load(1), format=_FMT)
                for k in range(2, K, 2):
                    pa, pb = plsc.unpack(load(k) + load(k + 1), format=_FMT)
                    sa = sa + pa
                    sb = sb + pb
                packed = plsc.pack(sa, sb, format=_FMT)
                agg_v[node, pl.ds(col, LANES)] = plsc.bitcast(packed, jnp.int32)
            return 0

        lax.fori_loop(0, C, node_body, 0)

    for b in range(NBUF - 1):
        issue(0, b, b)

    def group_body(i, _):
        c0 = i * NBUF
        for b in range(NBUF):
            c = c0 + b
            wait(i, b, b)
            slot = (b + NBUF - 1) % NBUF
            issue(i if b == 0 else i + 1, slot, slot)
            compute_chunk(c, b)
        return 0

    lax.fori_loop(0, GROUPS - 1, group_body, 0)
    g = GROUPS - 1
    c0 = g * NBUF
    for b in range(NBUF):
        c = c0 + b
        wait(g, b, b)
        if b == 0:
            issue(g, NBUF - 1, NBUF - 1)
        compute_chunk(c, b)
    pltpu.sync_copy(agg_v, out_hbm.at[wid])


@functools.cache
def _agg_call():
    mesh = plsc.VectorSubcoreMesh(core_axis_name="c", subcore_axis_name="s")
    return pl.kernel(
        _agg_body,
        out_type=jax.ShapeDtypeStruct((NW, NPW, DW), jnp.int32),
        mesh=mesh,
        scratch_types=[
            pltpu.VMEM((GROUPS, NBUF * CK), jnp.int32),
            pltpu.VMEM((NBUF, CK, DW), jnp.int32),
            pltpu.VMEM((NPW, DW), jnp.int32),
            pltpu.SemaphoreType.DMA,
        ],
        compiler_params=pltpu.CompilerParams(needs_layout_passes=False, skip_device_barrier=True),
    )


RT = 2000


def _mlp_body(eps_ref, xi_ref, agg_ref, w1e_ref, w1o_ref, b1_ref,
              w2_ref, b2_ref, o_ref):
    def halves(w):
        return (lax.bitcast_convert_type(w << 16, jnp.float32),
                lax.bitcast_convert_type(w & jnp.int32(-65536), jnp.float32))

    s = 1.0 + eps_ref[0]
    lx, hx = halves(xi_ref[...])
    la, ha = halves(agg_ref[...])
    lc = (s * lx + la).astype(jnp.bfloat16)
    hc = (s * hx + ha).astype(jnp.bfloat16)
    h1 = jnp.dot(lc, w1e_ref[...], preferred_element_type=jnp.float32)
    h1 = h1 + jnp.dot(hc, w1o_ref[...], preferred_element_type=jnp.float32)
    h1 = jnp.maximum(h1 + b1_ref[...], 0.0)
    o_ref[...] = jnp.dot(h1, w2_ref[...], preferred_element_type=jnp.float32) + b2_ref[...]


@functools.cache
def _mlp_call():
    return pl.pallas_call(
        _mlp_body,
        grid=(N // RT,),
        in_specs=[
            pl.BlockSpec(memory_space=pltpu.SMEM),
            pl.BlockSpec((RT, DW), lambda i: (i, 0)),
            pl.BlockSpec((RT, DW), lambda i: (i, 0)),
            pl.BlockSpec((DW, D), lambda i: (0, 0)),
            pl.BlockSpec((DW, D), lambda i: (0, 0)),
            pl.BlockSpec((1, D), lambda i: (0, 0)),
            pl.BlockSpec((D, D), lambda i: (0, 0)),
            pl.BlockSpec((1, D), lambda i: (0, 0)),
        ],
        out_specs=pl.BlockSpec((RT, D), lambda i: (i, 0)),
        out_shape=jax.ShapeDtypeStruct((N, D), jnp.float32),
    )


def kernel(x, neigh, eps, W1, b1, W2, b2):
    x2d = x[0]
    idx = neigh.astype(jnp.int32)
    pad_idx = (jnp.arange((NP - N) * K, dtype=jnp.int32) % N).reshape(NP - N, K)
    idx = jnp.concatenate([idx, pad_idx], axis=0)
    idx = idx.reshape(NW, GROUPS, NBUF * CK)
    y = lax.bitcast_convert_type(x2d, jnp.uint32)
    r = (y + jnp.uint32(0x8000)) >> 16
    xi = lax.bitcast_convert_type(r[:, :DW] | (r[:, DW:] << 16), jnp.int32)
    agg_i = _agg_call()(xi, idx).reshape(NP, DW)
    eps_arr = jnp.reshape(eps, (1,)).astype(jnp.float32)
    out = _mlp_call()(eps_arr, xi, agg_i,
                      W1[:DW].astype(jnp.bfloat16), W1[DW:].astype(jnp.bfloat16),
                      jnp.reshape(b1, (1, D)), W2, jnp.reshape(b2, (1, D)))
    return out[None]
